# Initial kernel scaffold; baseline (speedup 1.0000x reference)
import functools

import jax
import jax.numpy as jnp
from jax import lax
from jax.experimental import pallas as pl
from jax.experimental.pallas import tpu as pltpu
from jax.experimental.pallas import tpu_sc as plsc

N = 10000
D = 128
E = 320000
NC = 2
NS = 16
NW = NC * NS

CH = 128
NCHUNK = 79
EPW = NCHUNK * CH
E_PAD = NW * EPW
EPW_DEG = E // NW
ROWS_PT = N // NS
N_ACC = 10008
N_HIST = 10240

_MESH = plsc.VectorSubcoreMesh(
    core_axis_name="c", subcore_axis_name="s", num_cores=NC, num_subcores=NS
)


@functools.partial(
    pl.kernel,
    out_type=jax.ShapeDtypeStruct((NW, N_HIST), jnp.float32),
    mesh=_MESH,
    scratch_types=[
        pltpu.VMEM((N_HIST,), jnp.float32),
        pltpu.VMEM((EPW_DEG,), jnp.int32),
    ],
)
def _deg_kernel(dst_hbm, out_hbm, hist, dstv):
    cid = lax.axis_index("c")
    sid = lax.axis_index("s")
    wid = sid * NC + cid

    zeros16 = jnp.zeros((16,), jnp.float32)

    def zbody(i, c):
        hist[pl.ds(i * 16, 16)] = zeros16
        return c

    lax.fori_loop(0, N_HIST // 16, zbody, 0)

    off = pl.multiple_of(wid * EPW_DEG, 8)
    pltpu.sync_copy(dst_hbm.at[pl.ds(off, EPW_DEG)], dstv)

    ones16 = jnp.ones((16,), jnp.float32)

    def body(i, c):
        idx = dstv[pl.ds(i * 16, 16)]
        plsc.addupdate_scatter(hist, [idx], ones16)
        return c

    lax.fori_loop(0, EPW_DEG // 16, body, 0)
    pltpu.sync_copy(hist, out_hbm.at[wid])


@functools.partial(
    pl.kernel,
    out_type=jax.ShapeDtypeStruct((NC, N, D), jnp.float32),
    mesh=_MESH,
    scratch_types=[
        pltpu.VMEM_SHARED((N_ACC, D), jnp.float32),
        pltpu.VMEM((NCHUNK, CH), jnp.int32),
        pltpu.VMEM((NCHUNK, CH), jnp.int32),
        pltpu.VMEM((CH, D), jnp.float32),
        pltpu.SemaphoreType.DMA,
    ],
)
def _seg_kernel(xs_hbm, srcp_hbm, dstp_hbm, out_hbm, acc, srcv, dstv, rows, sem):
    cid = lax.axis_index("c")
    sid = lax.axis_index("s")
    wid = sid * NC + cid

    pltpu.sync_copy(
        xs_hbm.at[pl.ds(sid * ROWS_PT, ROWS_PT)],
        acc.at[pl.ds(sid * ROWS_PT, ROWS_PT)],
    )
    pltpu.sync_copy(srcp_hbm.at[wid], srcv)
    pltpu.sync_copy(dstp_hbm.at[wid], dstv)
    plsc.subcore_barrier()

    def body(g, c):
        pltpu.async_copy(xs_hbm.at[srcv.at[g]], rows, sem).wait()
        pltpu.sync_copy(rows, acc.at[dstv.at[g]], add=True)
        return c

    lax.fori_loop(0, NCHUNK, body, 0)
    plsc.subcore_barrier()

    pltpu.sync_copy(
        acc.at[pl.ds(sid * ROWS_PT, ROWS_PT)],
        out_hbm.at[cid, pl.ds(sid * ROWS_PT, ROWS_PT)],
    )


def _dinv_body(hists_ref, out_ref):
    deg = jnp.sum(hists_ref[...], axis=0) + 1.0
    out_ref[...] = lax.rsqrt(deg)


_dinv_call = pl.pallas_call(
    _dinv_body,
    out_shape=jax.ShapeDtypeStruct((N_HIST // 128, 128), jnp.float32),
)

BM = 2000


def _mm_body(x_ref, w_ref, dinv_ref, o_ref):
    xw = jnp.dot(x_ref[...], w_ref[...], preferred_element_type=jnp.float32)
    o_ref[...] = xw * dinv_ref[...]


_mm_call = pl.pallas_call(
    _mm_body,
    grid=(N // BM,),
    in_specs=[
        pl.BlockSpec((BM, D), lambda i: (i, 0)),
        pl.BlockSpec((D, D), lambda i: (0, 0)),
        pl.BlockSpec((BM, 1), lambda i: (i, 0)),
    ],
    out_specs=pl.BlockSpec((BM, D), lambda i: (i, 0)),
    out_shape=jax.ShapeDtypeStruct((N, D), jnp.float32),
)


def _post_body(p0_ref, p1_ref, xs_ref, dinv_ref, xin_ref, b_ref, g_ref, beta_ref, o_ref):
    agg = p0_ref[...] + p1_ref[...] - xs_ref[...]
    h = agg * dinv_ref[...] + b_ref[...]
    h = jnp.maximum(h, 0.0) + xin_ref[...]
    mu = jnp.mean(h, axis=-1, keepdims=True)
    d = h - mu
    var = jnp.mean(d * d, axis=-1, keepdims=True)
    o_ref[...] = d * lax.rsqrt(var + 1e-5) * g_ref[...] + beta_ref[...]


_post_call = pl.pallas_call(
    _post_body,
    grid=(N // BM,),
    in_specs=[
        pl.BlockSpec((BM, D), lambda i: (i, 0)),
        pl.BlockSpec((BM, D), lambda i: (i, 0)),
        pl.BlockSpec((BM, D), lambda i: (i, 0)),
        pl.BlockSpec((BM, 1), lambda i: (i, 0)),
        pl.BlockSpec((BM, D), lambda i: (i, 0)),
        pl.BlockSpec((1, D), lambda i: (0, 0)),
        pl.BlockSpec((1, D), lambda i: (0, 0)),
        pl.BlockSpec((1, D), lambda i: (0, 0)),
    ],
    out_specs=pl.BlockSpec((BM, D), lambda i: (i, 0)),
    out_shape=jax.ShapeDtypeStruct((N, D), jnp.float32),
)


@jax.jit
def _run(x, edge_index, Ws, bs, gammas, betas):
    src = edge_index[0]
    dst = edge_index[1]
    pad = E_PAD - E
    srcp = jnp.concatenate([src, jnp.zeros((pad,), jnp.int32)]).reshape(NW, NCHUNK, CH)
    dstp = jnp.concatenate([dst, jnp.full((pad,), N, jnp.int32)]).reshape(NW, NCHUNK, CH)

    hists = _deg_kernel(dst)
    dinv2d = _dinv_call(hists.reshape(NW, N_HIST // 128, 128))
    dinv_col = dinv2d.reshape(-1)[:N][:, None]

    for i in range(3):
        xs = _mm_call(x, Ws[i], dinv_col)
        parts = _seg_kernel(xs, srcp, dstp)
        x = _post_call(
            parts[0], parts[1], xs, dinv_col, x,
            bs[i][None, :], gammas[i][None, :], betas[i][None, :],
        )
    return x


def kernel(x, edge_index, Ws, bs, gammas, betas):
    return _run(x, edge_index, Ws, bs, gammas, betas)


# speedup vs baseline: 9.8774x; 9.8774x over previous
import functools

import jax
import jax.numpy as jnp
from jax import lax
from jax.experimental import pallas as pl
from jax.experimental.pallas import tpu as pltpu
from jax.experimental.pallas import tpu_sc as plsc

N = 10000
D = 128
E = 320000
NC = 2
NS = 16
NW = NC * NS

CH = 128
NCHUNK = 79
EPW = NCHUNK * CH
E_PAD = NW * EPW
EPW_DEG = E // NW
ROWS_PT = 624
ROW_TAIL = N - NS * ROWS_PT
N_ACC = 10008
N_HIST = 10240

@functools.cache
def _sc_kernels():
    mesh = plsc.VectorSubcoreMesh(
        core_axis_name="c", subcore_axis_name="s", num_cores=NC, num_subcores=NS
    )
    sc_params = pltpu.CompilerParams(needs_layout_passes=False)
    deg_kernel = functools.partial(
        pl.kernel,
        out_type=jax.ShapeDtypeStruct((NW, N_HIST), jnp.float32),
        mesh=mesh,
        compiler_params=sc_params,
        scratch_types=[
            pltpu.VMEM((N_HIST,), jnp.float32),
            pltpu.VMEM((EPW_DEG,), jnp.int32),
        ],
    )(_deg_body)
    seg_kernel = functools.partial(
        pl.kernel,
        out_type=jax.ShapeDtypeStruct((NC, N, D), jnp.float32),
        mesh=mesh,
        compiler_params=sc_params,
        scratch_types=[
            pltpu.VMEM_SHARED((N_ACC, D), jnp.float32),
            pltpu.VMEM((NCHUNK, CH), jnp.int32),
            pltpu.VMEM((NCHUNK, CH), jnp.int32),
            pltpu.VMEM((CH, D), jnp.float32),
            pltpu.SemaphoreType.DMA,
        ],
    )(_seg_body)
    return deg_kernel, seg_kernel


def _deg_body(dst_hbm, out_hbm, hist, dstv):
    cid = lax.axis_index("c")
    sid = lax.axis_index("s")
    wid = sid * NC + cid

    zeros16 = jnp.zeros((16,), jnp.float32)

    def zbody(i, c):
        hist[pl.ds(i * 16, 16)] = zeros16
        return c

    lax.fori_loop(0, N_HIST // 16, zbody, 0)

    off = pl.multiple_of(wid * EPW_DEG, 8)
    pltpu.sync_copy(dst_hbm.at[pl.ds(off, EPW_DEG)], dstv)

    ones16 = jnp.ones((16,), jnp.float32)

    def body(i, c):
        idx = dstv[pl.ds(i * 16, 16)]
        plsc.addupdate_scatter(hist, [idx], ones16)
        return c

    lax.fori_loop(0, EPW_DEG // 16, body, 0)
    pltpu.sync_copy(hist, out_hbm.at[wid])


def _seg_body(xs_hbm, srcp_hbm, dstp_hbm, out_hbm, acc, srcv, dstv, rows, sem):
    cid = lax.axis_index("c")
    sid = lax.axis_index("s")
    wid = sid * NC + cid

    r0 = pl.multiple_of(sid * ROWS_PT, 8)
    pltpu.sync_copy(xs_hbm.at[pl.ds(r0, ROWS_PT)], acc.at[pl.ds(r0, ROWS_PT)])

    @pl.when(sid == 0)
    def _():
        t0 = NS * ROWS_PT
        pltpu.sync_copy(
            xs_hbm.at[pl.ds(t0, ROW_TAIL)], acc.at[pl.ds(t0, ROW_TAIL)]
        )

    pltpu.sync_copy(srcp_hbm.at[wid], srcv)
    pltpu.sync_copy(dstp_hbm.at[wid], dstv)
    plsc.subcore_barrier()

    def body(g, c):
        pltpu.async_copy(xs_hbm.at[srcv.at[g]], rows, sem).wait()
        pltpu.sync_copy(rows, acc.at[dstv.at[g]], add=True)
        return c

    lax.fori_loop(0, NCHUNK, body, 0)
    plsc.subcore_barrier()

    pltpu.sync_copy(
        acc.at[pl.ds(r0, ROWS_PT)], out_hbm.at[cid, pl.ds(r0, ROWS_PT)]
    )

    @pl.when(sid == 0)
    def _():
        t0 = NS * ROWS_PT
        pltpu.sync_copy(
            acc.at[pl.ds(t0, ROW_TAIL)], out_hbm.at[cid, pl.ds(t0, ROW_TAIL)]
        )


def _dinv_body(hists_ref, out_ref):
    deg = jnp.sum(hists_ref[...], axis=0) + 1.0
    out_ref[...] = lax.rsqrt(deg)


_dinv_call = pl.pallas_call(
    _dinv_body,
    out_shape=jax.ShapeDtypeStruct((N_HIST // 128, 128), jnp.float32),
)

BM = 2000


def _mm_body(x_ref, w_ref, dinv_ref, o_ref):
    xw = jnp.dot(x_ref[...], w_ref[...], preferred_element_type=jnp.float32)
    o_ref[...] = xw * dinv_ref[...]


_mm_call = pl.pallas_call(
    _mm_body,
    grid=(N // BM,),
    in_specs=[
        pl.BlockSpec((BM, D), lambda i: (i, 0)),
        pl.BlockSpec((D, D), lambda i: (0, 0)),
        pl.BlockSpec((BM, 1), lambda i: (i, 0)),
    ],
    out_specs=pl.BlockSpec((BM, D), lambda i: (i, 0)),
    out_shape=jax.ShapeDtypeStruct((N, D), jnp.float32),
)


def _post_body(p0_ref, p1_ref, xs_ref, dinv_ref, xin_ref, b_ref, g_ref, beta_ref, o_ref):
    agg = p0_ref[...] + p1_ref[...] - xs_ref[...]
    h = agg * dinv_ref[...] + b_ref[...]
    h = jnp.maximum(h, 0.0) + xin_ref[...]
    mu = jnp.mean(h, axis=-1, keepdims=True)
    d = h - mu
    var = jnp.mean(d * d, axis=-1, keepdims=True)
    o_ref[...] = d * lax.rsqrt(var + 1e-5) * g_ref[...] + beta_ref[...]


_post_call = pl.pallas_call(
    _post_body,
    grid=(N // BM,),
    in_specs=[
        pl.BlockSpec((BM, D), lambda i: (i, 0)),
        pl.BlockSpec((BM, D), lambda i: (i, 0)),
        pl.BlockSpec((BM, D), lambda i: (i, 0)),
        pl.BlockSpec((BM, 1), lambda i: (i, 0)),
        pl.BlockSpec((BM, D), lambda i: (i, 0)),
        pl.BlockSpec((1, D), lambda i: (0, 0)),
        pl.BlockSpec((1, D), lambda i: (0, 0)),
        pl.BlockSpec((1, D), lambda i: (0, 0)),
    ],
    out_specs=pl.BlockSpec((BM, D), lambda i: (i, 0)),
    out_shape=jax.ShapeDtypeStruct((N, D), jnp.float32),
)


@jax.jit
def _run(x, edge_index, Ws, bs, gammas, betas):
    src = edge_index[0]
    dst = edge_index[1]
    pad = E_PAD - E
    srcp = jnp.concatenate([src, jnp.zeros((pad,), jnp.int32)]).reshape(NW, NCHUNK, CH)
    dstp = jnp.concatenate([dst, jnp.full((pad,), N, jnp.int32)]).reshape(NW, NCHUNK, CH)

    deg_kernel, seg_kernel = _sc_kernels()
    hists = deg_kernel(dst)
    dinv2d = _dinv_call(hists.reshape(NW, N_HIST // 128, 128))
    dinv_col = dinv2d.reshape(-1)[:N][:, None]

    for i in range(3):
        xs = _mm_call(x, Ws[i], dinv_col)
        parts = seg_kernel(xs, srcp, dstp)
        x = _post_call(
            parts[0], parts[1], xs, dinv_col, x,
            bs[i][None, :], gammas[i][None, :], betas[i][None, :],
        )
    return x


def kernel(x, edge_index, Ws, bs, gammas, betas):
    return _run(x, edge_index, Ws, bs, gammas, betas)
